# Initial kernel scaffold; baseline (speedup 1.0000x reference)
import jax
import jax.numpy as jnp
from jax.experimental import pallas as pl


def kernel(x, edge_index, edge_attr, W1, gamma1, beta1, W2, gamma2, beta2, epsilon):
    raise NotImplementedError("write your pallas kernel here")


# speedup vs baseline: 6.9542x; 6.9542x over previous
import functools

import jax
import jax.numpy as jnp
from jax import lax
from jax.experimental import pallas as pl
from jax.experimental.pallas import tpu as pltpu
from jax.experimental.pallas import tpu_sc as plsc

N = 10000
E = 320000
D = 128
H = 2 * D
BN_EPS = 1e-5

NC = 2
NS = 16
NW = NC * NS

N_PAD = 10112
ROWS_PER_TILE = N_PAD // NS
EDGES_PER_TILE = E // NW
CHUNK = 80
NCHUNKS = EDGES_PER_TILE // CHUNK


def _sc_body(x_hbm, src_hbm, dst_hbm, ea_hbm, zero_hbm, out_hbm,
             src_v, dst_v, ea_v, xr_v, acc_sh, sem):
    c = lax.axis_index("c")
    s = lax.axis_index("s")
    wid = s * jnp.int32(NC) + c

    row0 = s * jnp.int32(ROWS_PER_TILE)
    pltpu.sync_copy(zero_hbm, acc_sh.at[pl.ds(row0, ROWS_PER_TILE)])
    plsc.subcore_barrier()

    base_e = wid * jnp.int32(EDGES_PER_TILE)

    def chunk_body(g, carry):
        off = base_e + g * jnp.int32(CHUNK)
        pltpu.sync_copy(src_hbm.at[pl.ds(off, CHUNK)], src_v)
        pltpu.sync_copy(dst_hbm.at[pl.ds(off, CHUNK)], dst_v)
        pltpu.sync_copy(ea_hbm.at[pl.ds(off, CHUNK)], ea_v)
        pltpu.async_copy(x_hbm.at[src_v], xr_v, sem).wait()

        def row_body(r, cc):
            for k in range(D // 16):
                sl = pl.ds(k * 16, 16)
                xr_v[r, sl] = jnp.maximum(xr_v[r, sl] + ea_v[r, sl], 0.0)
            return cc

        lax.fori_loop(jnp.int32(0), jnp.int32(CHUNK), row_body, jnp.int32(0))
        pltpu.sync_copy(xr_v, acc_sh.at[dst_v], add=True)
        return carry

    lax.fori_loop(jnp.int32(0), jnp.int32(NCHUNKS), chunk_body, jnp.int32(0))
    plsc.subcore_barrier()

    rs = pl.ds(row0, ROWS_PER_TILE)
    pltpu.sync_copy(acc_sh.at[rs], out_hbm.at[c, rs])


@functools.cache
def _sc_scatter():
    return pl.kernel(
        _sc_body,
        mesh=plsc.VectorSubcoreMesh(core_axis_name="c", subcore_axis_name="s"),
        out_type=jax.ShapeDtypeStruct((NC, N_PAD, D), jnp.float32),
        scratch_types=[
            pltpu.VMEM((CHUNK,), jnp.int32),
            pltpu.VMEM((CHUNK,), jnp.int32),
            pltpu.VMEM((CHUNK, D), jnp.float32),
            pltpu.VMEM((CHUNK, D), jnp.float32),
            pltpu.VMEM_SHARED((N_PAD, D), jnp.float32),
            pltpu.SemaphoreType.DMA,
        ],
    )


def _tc_body(acc_ref, x_ref, w1_ref, g1_ref, b1_ref, w2_ref, g2_ref,
             b2_ref, eps_ref, o_ref):
    nn = acc_ref[0][:N, :] + acc_ref[1][:N, :]
    h = nn + (1.0 + eps_ref[0, 0]) * x_ref[...]
    h = jnp.dot(h, w1_ref[...], preferred_element_type=jnp.float32,
                precision=lax.Precision.HIGHEST)
    mu = jnp.mean(h, axis=0, keepdims=True)
    d = h - mu
    var = jnp.mean(d * d, axis=0, keepdims=True)
    h = d * lax.rsqrt(var + BN_EPS) * g1_ref[...] + b1_ref[...]
    h = jnp.maximum(h, 0.0)
    h = jnp.dot(h, w2_ref[...], preferred_element_type=jnp.float32,
                precision=lax.Precision.HIGHEST)
    mu = jnp.mean(h, axis=0, keepdims=True)
    d = h - mu
    var = jnp.mean(d * d, axis=0, keepdims=True)
    h = d * lax.rsqrt(var + BN_EPS) * g2_ref[...] + b2_ref[...]
    o_ref[...] = jnp.maximum(h, 0.0)


_tc_mlp = pl.pallas_call(
    _tc_body,
    out_shape=jax.ShapeDtypeStruct((N, D), jnp.float32),
)


@jax.jit
def kernel(x, edge_index, edge_attr, W1, gamma1, beta1, W2, gamma2, beta2,
           epsilon):
    out_dtype = jnp.result_type(x.dtype, W1.dtype, W2.dtype)
    src = edge_index[0].astype(jnp.int32)
    dst = edge_index[1].astype(jnp.int32)
    zero = jnp.zeros((ROWS_PER_TILE, D), jnp.float32)
    acc = _sc_scatter()(x, src, dst, edge_attr, zero)
    out = _tc_mlp(acc, x, W1.astype(jnp.float32),
                  gamma1.reshape(1, H).astype(jnp.float32),
                  beta1.reshape(1, H).astype(jnp.float32),
                  W2.astype(jnp.float32),
                  gamma2.reshape(1, D).astype(jnp.float32),
                  beta2.reshape(1, D).astype(jnp.float32),
                  epsilon.reshape(1, 1).astype(jnp.float32))
    return out.astype(out_dtype)


# speedup vs baseline: 14.4417x; 2.0767x over previous
import functools

import jax
import jax.numpy as jnp
from jax import lax
from jax.experimental import pallas as pl
from jax.experimental.pallas import tpu as pltpu
from jax.experimental.pallas import tpu_sc as plsc

N = 10000
E = 320000
D = 128
H = 2 * D
BN_EPS = 1e-5

NC = 2
NS = 16
NW = NC * NS

N_PAD = 10112
ROWS_PER_TILE = N_PAD // NS
CHUNK = 80
TOTAL_CHUNKS = E // CHUNK
BASE_CHUNKS = TOTAL_CHUNKS // NW
EXTRA_CHUNKS = TOTAL_CHUNKS - BASE_CHUNKS * NW


def _sc_body(x_hbm, src_hbm, dst_hbm, ea_hbm, zero_hbm, out_hbm,
             src0, src1, dst0, dst1, ea0, ea1, xr0, xr1, acc_sh,
             s_src0, s_src1, s_dst0, s_dst1, s_io0, s_io1):
    c = lax.axis_index("c")
    s = lax.axis_index("s")
    wid = s * jnp.int32(NC) + c

    row0 = s * jnp.int32(ROWS_PER_TILE)
    pltpu.sync_copy(zero_hbm, acc_sh.at[pl.ds(row0, ROWS_PER_TILE)])
    plsc.subcore_barrier()

    start = wid * jnp.int32(BASE_CHUNKS) + jnp.minimum(wid, EXTRA_CHUNKS)
    count = jnp.int32(BASE_CHUNKS) + (wid < EXTRA_CHUNKS).astype(jnp.int32)

    P0 = (src0, dst0, ea0, xr0, s_src0, s_dst0, s_io0)
    P1 = (src1, dst1, ea1, xr1, s_src1, s_dst1, s_io1)

    def e_off(g):
        return (start + g) * jnp.int32(CHUNK)

    def issue_src(g, P):
        pltpu.async_copy(src_hbm.at[pl.ds(e_off(g), CHUNK)], P[0], P[4])

    def wait_src(P):
        pltpu.make_async_copy(src_hbm.at[pl.ds(0, CHUNK)], P[0], P[4]).wait()

    def issue_dst(g, P):
        pltpu.async_copy(dst_hbm.at[pl.ds(e_off(g), CHUNK)], P[1], P[5])

    def wait_dst(P):
        pltpu.make_async_copy(dst_hbm.at[pl.ds(0, CHUNK)], P[1], P[5]).wait()

    def issue_io(g, P):
        pltpu.async_copy(ea_hbm.at[pl.ds(e_off(g), CHUNK)], P[2], P[6])
        pltpu.async_copy(x_hbm.at[P[0]], P[3], P[6])

    def wait_io(P):
        pltpu.make_async_copy(ea_hbm.at[pl.ds(0, CHUNK)], P[2], P[6]).wait()
        pltpu.make_async_copy(x_hbm.at[pl.ds(0, CHUNK)], P[3], P[6]).wait()

    def do_chunk(g, cur, nxt):
        @pl.when(g + jnp.int32(1) < count)
        def _():
            wait_src(nxt)
            issue_io(g + jnp.int32(1), nxt)
            issue_dst(g + jnp.int32(1), nxt)

        wait_io(cur)

        @pl.when(g + jnp.int32(2) < count)
        def _():
            issue_src(g + jnp.int32(2), cur)

        ea_v, xr_v = cur[2], cur[3]

        def row_body(r, cc):
            for k in range(D // 16):
                sl = pl.ds(k * 16, 16)
                xr_v[r, sl] = jnp.maximum(xr_v[r, sl] + ea_v[r, sl], 0.0)
            return cc

        lax.fori_loop(jnp.int32(0), jnp.int32(CHUNK), row_body, jnp.int32(0))
        wait_dst(cur)
        pltpu.sync_copy(xr_v, acc_sh.at[cur[1]], add=True)

    issue_src(jnp.int32(0), P0)
    issue_dst(jnp.int32(0), P0)
    wait_src(P0)
    issue_io(jnp.int32(0), P0)
    issue_src(jnp.int32(1), P1)

    def pair_body(p, cc):
        g = p * jnp.int32(2)
        do_chunk(g, P0, P1)
        do_chunk(g + jnp.int32(1), P1, P0)
        return cc

    lax.fori_loop(jnp.int32(0), count // jnp.int32(2), pair_body, jnp.int32(0))

    @pl.when(count % jnp.int32(2) == jnp.int32(1))
    def _():
        do_chunk(count - jnp.int32(1), P0, P1)

    plsc.subcore_barrier()
    rs = pl.ds(row0, ROWS_PER_TILE)
    pltpu.sync_copy(acc_sh.at[rs], out_hbm.at[c, rs])


@functools.cache
def _sc_scatter():
    return pl.kernel(
        _sc_body,
        mesh=plsc.VectorSubcoreMesh(core_axis_name="c", subcore_axis_name="s"),
        out_type=jax.ShapeDtypeStruct((NC, N_PAD, D), jnp.float32),
        scratch_types=[
            pltpu.VMEM((CHUNK,), jnp.int32),
            pltpu.VMEM((CHUNK,), jnp.int32),
            pltpu.VMEM((CHUNK,), jnp.int32),
            pltpu.VMEM((CHUNK,), jnp.int32),
            pltpu.VMEM((CHUNK, D), jnp.float32),
            pltpu.VMEM((CHUNK, D), jnp.float32),
            pltpu.VMEM((CHUNK, D), jnp.float32),
            pltpu.VMEM((CHUNK, D), jnp.float32),
            pltpu.VMEM_SHARED((N_PAD, D), jnp.float32),
            pltpu.SemaphoreType.DMA,
            pltpu.SemaphoreType.DMA,
            pltpu.SemaphoreType.DMA,
            pltpu.SemaphoreType.DMA,
            pltpu.SemaphoreType.DMA,
            pltpu.SemaphoreType.DMA,
        ],
    )


def _tc_body(acc_ref, x_ref, w1_ref, g1_ref, b1_ref, w2_ref, g2_ref,
             b2_ref, eps_ref, o_ref):
    nn = acc_ref[0][:N, :] + acc_ref[1][:N, :]
    h = nn + (1.0 + eps_ref[0, 0]) * x_ref[...]
    h = jnp.dot(h, w1_ref[...], preferred_element_type=jnp.float32,
                precision=lax.Precision.HIGHEST)
    mu = jnp.mean(h, axis=0, keepdims=True)
    d = h - mu
    var = jnp.mean(d * d, axis=0, keepdims=True)
    h = d * lax.rsqrt(var + BN_EPS) * g1_ref[...] + b1_ref[...]
    h = jnp.maximum(h, 0.0)
    h = jnp.dot(h, w2_ref[...], preferred_element_type=jnp.float32,
                precision=lax.Precision.HIGHEST)
    mu = jnp.mean(h, axis=0, keepdims=True)
    d = h - mu
    var = jnp.mean(d * d, axis=0, keepdims=True)
    h = d * lax.rsqrt(var + BN_EPS) * g2_ref[...] + b2_ref[...]
    o_ref[...] = jnp.maximum(h, 0.0)


_tc_mlp = pl.pallas_call(
    _tc_body,
    out_shape=jax.ShapeDtypeStruct((N, D), jnp.float32),
)


@jax.jit
def kernel(x, edge_index, edge_attr, W1, gamma1, beta1, W2, gamma2, beta2,
           epsilon):
    out_dtype = jnp.result_type(x.dtype, W1.dtype, W2.dtype)
    src = edge_index[0].astype(jnp.int32)
    dst = edge_index[1].astype(jnp.int32)
    zero = jnp.zeros((ROWS_PER_TILE, D), jnp.float32)
    acc = _sc_scatter()(x, src, dst, edge_attr, zero)
    out = _tc_mlp(acc, x, W1.astype(jnp.float32),
                  gamma1.reshape(1, H).astype(jnp.float32),
                  beta1.reshape(1, H).astype(jnp.float32),
                  W2.astype(jnp.float32),
                  gamma2.reshape(1, D).astype(jnp.float32),
                  beta2.reshape(1, D).astype(jnp.float32),
                  epsilon.reshape(1, 1).astype(jnp.float32))
    return out.astype(out_dtype)
